# baseline (speedup 1.0000x reference)
import functools

import jax
import jax.numpy as jnp
from jax import lax
from jax.experimental import pallas as pl
from jax.experimental.pallas import tpu as pltpu
from jax.experimental.pallas import tpu_sc as plsc

_MAX_LEN = 8192
_D = 1024
_NC = 2
_NS = 16
_NW = _NC * _NS
_A_T = _MAX_LEN // _NW
_CHUNK = 32
_NCHA = _A_T // _CHUNK
_NBUF = 3


def _sc_gather(table):
    mesh = plsc.VectorSubcoreMesh(
        core_axis_name="c", subcore_axis_name="s",
        num_cores=_NC, num_subcores=_NS)

    @functools.partial(
        pl.kernel,
        out_type=jax.ShapeDtypeStruct((_MAX_LEN, _D), jnp.float32),
        mesh=mesh,
        scratch_types=(
            [pltpu.VMEM((_A_T,), jnp.int32)]
            + [pltpu.VMEM((_CHUNK, _D), jnp.float32) for _ in range(_NBUF)]
            + [pltpu.SemaphoreType.DMA for _ in range(2 * _NBUF)]
        ),
    )
    def k(table_hbm, out_hbm, idx_v, *rest):
        bufs = rest[:_NBUF]
        gsems = rest[_NBUF:2 * _NBUF]
        wsems = rest[2 * _NBUF:]

        wid = lax.axis_index("c") * _NS + lax.axis_index("s")
        base = wid * _A_T

        lane = lax.iota(jnp.int32, 16)
        for i in range(_A_T // 16):
            idx_v[pl.ds(16 * i, 16)] = lane + (base + 16 * i)

        def gather(c, s):
            return pltpu.async_copy(
                table_hbm.at[idx_v.at[pl.ds(c * _CHUNK, _CHUNK)]],
                bufs[s], gsems[s])

        def put(c, s):
            return pltpu.async_copy(
                bufs[s], out_hbm.at[pl.ds(base + c * _CHUNK, _CHUNK)],
                wsems[s])

        lead = _NBUF - 1
        g = [None] * _NCHA
        w = [None] * _NCHA
        unwaited = set()
        for c in range(min(lead, _NCHA)):
            g[c] = gather(c, c % _NBUF)
        for c in range(_NCHA):
            g[c].wait()
            w[c] = put(c, c % _NBUF)
            unwaited.add(c)
            n = c + lead
            if n < _NCHA:
                if c >= 1:
                    w[c - 1].wait()
                    unwaited.discard(c - 1)
                g[n] = gather(n, n % _NBUF)
        for c in sorted(unwaited):
            w[c].wait()

    return k(table)


def kernel(T, pos_emb):
    del T
    out = _sc_gather(pos_emb)
    return out[None, :, :]


# speedup vs baseline: 1.0513x; 1.0513x over previous
import functools

import jax
import jax.numpy as jnp
from jax import lax
from jax.experimental import pallas as pl
from jax.experimental.pallas import tpu as pltpu
from jax.experimental.pallas import tpu_sc as plsc

_MAX_LEN = 8192
_D = 1024
_NC = 2
_NS = 16
_NW = _NC * _NS
_A_T = _MAX_LEN // _NW
_CHUNK = 8
_NCHA = _A_T // _CHUNK
_NBUF = 14


def _sc_gather(table):
    mesh = plsc.VectorSubcoreMesh(
        core_axis_name="c", subcore_axis_name="s",
        num_cores=_NC, num_subcores=_NS)

    @functools.partial(
        pl.kernel,
        out_type=jax.ShapeDtypeStruct((_MAX_LEN, _D), jnp.float32),
        mesh=mesh,
        scratch_types=(
            [pltpu.VMEM((_A_T,), jnp.int32)]
            + [pltpu.VMEM((_CHUNK, _D), jnp.float32) for _ in range(_NBUF)]
            + [pltpu.SemaphoreType.DMA for _ in range(2 * _NBUF)]
        ),
    )
    def k(table_hbm, out_hbm, idx_v, *rest):
        bufs = rest[:_NBUF]
        gsems = rest[_NBUF:2 * _NBUF]
        wsems = rest[2 * _NBUF:]

        wid = lax.axis_index("c") * _NS + lax.axis_index("s")
        base = wid * _A_T

        lane = lax.iota(jnp.int32, 16)
        for i in range(_A_T // 16):
            idx_v[pl.ds(16 * i, 16)] = lane + (base + 16 * i)

        def gather(c, s):
            return pltpu.async_copy(
                table_hbm.at[idx_v.at[pl.ds(c * _CHUNK, _CHUNK)]],
                bufs[s], gsems[s])

        def put(c, s):
            return pltpu.async_copy(
                bufs[s], out_hbm.at[pl.ds(base + c * _CHUNK, _CHUNK)],
                wsems[s])

        lead = _NBUF - 1
        g = [None] * _NCHA
        w = [None] * _NCHA
        unwaited = set()
        for c in range(min(lead, _NCHA)):
            g[c] = gather(c, c % _NBUF)
        for c in range(_NCHA):
            g[c].wait()
            w[c] = put(c, c % _NBUF)
            unwaited.add(c)
            n = c + lead
            if n < _NCHA:
                if c >= 1:
                    w[c - 1].wait()
                    unwaited.discard(c - 1)
                g[n] = gather(n, n % _NBUF)
        for c in sorted(unwaited):
            w[c].wait()

    return k(table)


def kernel(T, pos_emb):
    del T
    out = _sc_gather(pos_emb)
    return out[None, :, :]
